# baseline (speedup 1.0000x reference)
import functools

import jax
import jax.numpy as jnp
from jax import lax
from jax.experimental import pallas as pl
from jax.experimental.pallas import tpu as pltpu
from jax.experimental.pallas import tpu_sc as plsc

B, T, N, E = 2, 4, 10000, 320000
D = 128
NS = 16
NC = 2
NPAD = 10240
ZCH = NPAD // NS
CK = 80
CHT = 250
EPAD = NS * CHT * CK
NBUF = 4
NTRASH = 10000

_mesh = plsc.VectorSubcoreMesh(core_axis_name="c", subcore_axis_name="s")


DNB = 5


def _deg_body(col_hbm, deg_hbm, c0, c1, c2, c3, c4, c5, onesv, zbuf,
              si0, si1, si2, si3, si4, si5,
              ss0, ss1, ss2, ss3, ss4, ss5, deg_sh):
    c = lax.axis_index("c")
    s = lax.axis_index("s")
    colv = (c0, c1, c2, c3, c4, c5)
    si = (si0, si1, si2, si3, si4, si5)
    ss = (ss0, ss1, ss2, ss3, ss4, ss5)
    for j in range(ZCH // 16):
        zbuf[pl.ds(j * 16, 16)] = jnp.zeros((16,), jnp.float32)
    for j in range(CK // 16):
        onesv[pl.ds(j * 16, 16)] = jnp.ones((16,), jnp.float32)
    onesv[pl.ds(CK - 16, 16)] = jnp.ones((16,), jnp.float32)
    pltpu.sync_copy(zbuf, deg_sh.at[pl.ds(s * ZCH, ZCH)])
    plsc.subcore_barrier()
    ept = EPAD // (NS * NC)
    base = (s * NC + c) * ept

    def block(outer, carry):
        idp = []
        for b in range(DNB):
            @pl.when(outer > 0)
            def _():
                pltpu.make_async_copy(onesv.at[pl.ds(0, CK)],
                                      deg_sh.at[colv[b]], ss[b]).wait()

            off = base + (outer * DNB + b) * CK
            idp.append(pltpu.async_copy(col_hbm.at[pl.ds(off, CK)],
                                        colv[b], si[b]))
        for b in range(DNB):
            idp[b].wait()
            pltpu.async_copy(onesv.at[pl.ds(0, CK)], deg_sh.at[colv[b]],
                             ss[b], add=True)
        return carry

    lax.fori_loop(0, ept // (CK * DNB), block, 0)
    for b in range(DNB):
        pltpu.make_async_copy(onesv.at[pl.ds(0, CK)], deg_sh.at[colv[b]],
                              ss[b]).wait()
    plsc.subcore_barrier()
    pltpu.sync_copy(deg_sh.at[pl.ds(s * ZCH, ZCH)],
                    deg_hbm.at[pl.ds(c * NPAD + s * ZCH, ZCH)])


_deg = pl.kernel(
    _deg_body,
    out_type=jax.ShapeDtypeStruct((NC * NPAD,), jnp.float32),
    mesh=_mesh,
    scratch_types=(
        [pltpu.VMEM((CK,), jnp.int32) for _ in range(6)]
        + [pltpu.VMEM((CK,), jnp.float32), pltpu.VMEM((ZCH,), jnp.float32)]
        + [pltpu.SemaphoreType.DMA for _ in range(12)]
        + [pltpu.VMEM_SHARED((NPAD,), jnp.float32)]
    ),
)


def _make_prop(nt):
    def body(xs_hbm, rowb_hbm, col_hbm, out_hbm,
             r0, r1, r2, r3, c0, c1, c2, c3, g0, g1, g2, g3,
             si0, si1, si2, si3, sg0, sg1, sg2, sg3,
             ss0, ss1, ss2, ss3, out_sh):
        c = lax.axis_index("c")
        s = lax.axis_index("s")
        rowv = (r0, r1, r2, r3)
        colv = (c0, c1, c2, c3)
        gbuf = (g0, g1, g2, g3)
        si = (si0, si1, si2, si3)
        sg = (sg0, sg1, sg2, sg3)
        ss = (ss0, ss1, ss2, ss3)
        rpt = N // 10
        ept = CHT * CK
        base = s * ept

        for t in range(nt):
            @pl.when(s < 10)
            def _():
                pltpu.sync_copy(
                    xs_hbm.at[pl.ds(t * B * N + c * N + s * rpt, rpt)],
                    out_sh.at[pl.ds(s * rpt, rpt)])

            plsc.subcore_barrier()
            ibase = (t * NC + c) * EPAD + base

            def block(outer, carry):
                idp = []
                for b in range(NBUF):
                    @pl.when(outer > 0)
                    def _():
                        pltpu.make_async_copy(gbuf[b], out_sh.at[colv[b]],
                                              ss[b]).wait()

                    off = (outer * NBUF + b) * CK
                    i1 = pltpu.async_copy(
                        rowb_hbm.at[pl.ds(ibase + off, CK)], rowv[b], si[b])
                    i2 = pltpu.async_copy(
                        col_hbm.at[pl.ds(base + off, CK)], colv[b], si[b])
                    idp.append((i1, i2))
                gd = []
                for b in range(NBUF):
                    idp[b][0].wait()
                    idp[b][1].wait()
                    gd.append(pltpu.async_copy(xs_hbm.at[rowv[b]], gbuf[b],
                                               sg[b]))
                for b in range(NBUF):
                    gd[b].wait()
                    pltpu.async_copy(gbuf[b], out_sh.at[colv[b]], ss[b],
                                     add=True)
                return carry

            lax.fori_loop(0, CHT // NBUF, block, 0)
            for k in range(CHT - NBUF * (CHT // NBUF)):
                pltpu.make_async_copy(gbuf[k], out_sh.at[colv[k]],
                                      ss[k]).wait()
                off = (NBUF * (CHT // NBUF) + k) * CK
                pltpu.sync_copy(rowb_hbm.at[pl.ds(ibase + off, CK)], rowv[k])
                pltpu.sync_copy(col_hbm.at[pl.ds(base + off, CK)], colv[k])
                pltpu.sync_copy(xs_hbm.at[rowv[k]], gbuf[k])
                pltpu.async_copy(gbuf[k], out_sh.at[colv[k]], ss[k], add=True)
            for b in range(NBUF):
                pltpu.make_async_copy(gbuf[b], out_sh.at[colv[b]],
                                      ss[b]).wait()
            plsc.subcore_barrier()

            @pl.when(s < 10)
            def _():
                pltpu.sync_copy(
                    out_sh.at[pl.ds(s * rpt, rpt)],
                    out_hbm.at[pl.ds(t * B * N + c * N + s * rpt, rpt)])

    return pl.kernel(
        body,
        out_type=jax.ShapeDtypeStruct((nt * B * N, D), jnp.float32),
        mesh=_mesh,
        scratch_types=(
            [pltpu.VMEM((CK,), jnp.int32) for _ in range(NBUF)]
            + [pltpu.VMEM((CK,), jnp.int32) for _ in range(NBUF)]
            + [pltpu.VMEM((CK, D), jnp.float32) for _ in range(NBUF)]
            + [pltpu.SemaphoreType.DMA for _ in range(3 * NBUF)]
            + [pltpu.VMEM_SHARED((NTRASH, D), jnp.float32)]
        ),
    )


_prop = _make_prop(1)


def _dinv_body(deg_ref, out_ref):
    out_ref[...] = lax.rsqrt(deg_ref[0] + deg_ref[1] + 1.0)


def _dinv(degs):
    return pl.pallas_call(
        _dinv_body,
        out_shape=jax.ShapeDtypeStruct((NPAD // D, D), jnp.float32),
    )(degs)


def _scale_body(x_ref, d_ref, o_ref):
    o_ref[0, 0] = x_ref[0, 0] * d_ref[...]


def _scale(xT, dinvb, rb):
    return pl.pallas_call(
        _scale_body,
        grid=(T, B, N // rb),
        in_specs=[
            pl.BlockSpec((1, 1, rb, D), lambda t, b, i: (t, b, i, 0)),
            pl.BlockSpec((rb, D), lambda t, b, i: (i, 0)),
        ],
        out_specs=pl.BlockSpec((1, 1, rb, D), lambda t, b, i: (t, b, i, 0)),
        out_shape=jax.ShapeDtypeStruct((T, B, N, D), jnp.float32),
    )(xT, dinvb)


def _gate0_body(px, dv, wx, bx, bh, wf, bf, hn_o, hs_o, out_o):
    d = dv[...]
    a = (px[0] * d).astype(jnp.bfloat16)
    u = jnp.dot(a, wx[...], preferred_element_type=jnp.float32) + bx[...]
    v = bh[...]
    r = jax.nn.sigmoid(u[:, :D] + v[:, :D])
    z = jax.nn.sigmoid(u[:, D:2 * D] + v[:, D:2 * D])
    n = jnp.tanh(u[:, 2 * D:] + r * v[:, 2 * D:])
    hn = z * n
    hn_o[0] = hn
    hs_o[0] = hn * d
    out_o[0] = jnp.dot(hn.astype(jnp.bfloat16), wf[...],
                       preferred_element_type=jnp.float32) + bf[...]


def _gate0(Px, dinvb, Wx, bx, bh, Wf, bf, rb):
    node = pl.BlockSpec((1, rb, D), lambda b, i: (b, i, 0))
    return pl.pallas_call(
        _gate0_body,
        grid=(B, N // rb),
        in_specs=[
            node,
            pl.BlockSpec((rb, D), lambda b, i: (i, 0)),
            pl.BlockSpec((D, 3 * D), lambda b, i: (0, 0)),
            pl.BlockSpec((1, 3 * D), lambda b, i: (0, 0)),
            pl.BlockSpec((1, 3 * D), lambda b, i: (0, 0)),
            pl.BlockSpec((D, D), lambda b, i: (0, 0)),
            pl.BlockSpec((1, D), lambda b, i: (0, 0)),
        ],
        out_specs=[node, node, node],
        out_shape=[
            jax.ShapeDtypeStruct((B, N, D), jnp.float32),
            jax.ShapeDtypeStruct((B, N, D), jnp.float32),
            jax.ShapeDtypeStruct((B, N, D), jnp.float32),
        ],
    )(Px, dinvb, Wx, bx, bh, Wf, bf)


def _gate_body(px, ph, h, dv, wx, bx, wh, bh, wf, bf, hn_o, hs_o, out_o):
    d = dv[...]
    u = jnp.dot((px[0] * d).astype(jnp.bfloat16), wx[...],
                preferred_element_type=jnp.float32) + bx[...]
    v = jnp.dot((ph[0] * d).astype(jnp.bfloat16), wh[...],
                preferred_element_type=jnp.float32) + bh[...]
    r = jax.nn.sigmoid(u[:, :D] + v[:, :D])
    z = jax.nn.sigmoid(u[:, D:2 * D] + v[:, D:2 * D])
    n = jnp.tanh(u[:, 2 * D:] + r * v[:, 2 * D:])
    hn = (1.0 - z) * h[0] + z * n
    hn_o[0] = hn
    hs_o[0] = hn * d
    out_o[0] = jnp.dot(hn.astype(jnp.bfloat16), wf[...],
                       preferred_element_type=jnp.float32) + bf[...]


def _gate(Px, Ph, h, dinvb, Wx, bx, Wh, bh, Wf, bf, rb):
    node = pl.BlockSpec((1, rb, D), lambda b, i: (b, i, 0))
    return pl.pallas_call(
        _gate_body,
        grid=(B, N // rb),
        in_specs=[
            node, node, node,
            pl.BlockSpec((rb, D), lambda b, i: (i, 0)),
            pl.BlockSpec((D, 3 * D), lambda b, i: (0, 0)),
            pl.BlockSpec((1, 3 * D), lambda b, i: (0, 0)),
            pl.BlockSpec((D, 3 * D), lambda b, i: (0, 0)),
            pl.BlockSpec((1, 3 * D), lambda b, i: (0, 0)),
            pl.BlockSpec((D, D), lambda b, i: (0, 0)),
            pl.BlockSpec((1, D), lambda b, i: (0, 0)),
        ],
        out_specs=[node, node, node],
        out_shape=[
            jax.ShapeDtypeStruct((B, N, D), jnp.float32),
            jax.ShapeDtypeStruct((B, N, D), jnp.float32),
            jax.ShapeDtypeStruct((B, N, D), jnp.float32),
        ],
    )(Px, Ph, h, dinvb, Wx, bx, Wh, bh, Wf, bf)


def kernel(x, edge_index, Wxr, bxr, Wxz, bxz, Wxn, bxn,
           Whr, bhr, Whz, bhz, Whn, bhn, Wfc, bfc):
    row = edge_index[0]
    col = edge_index[1]
    rowb = jnp.concatenate([row, row + N])
    col3 = col
    Wxcat = jnp.concatenate([Wxr, Wxz, Wxn], axis=1).astype(jnp.bfloat16)
    Whcat = jnp.concatenate([Whr, Whz, Whn], axis=1).astype(jnp.bfloat16)
    bxcat = jnp.concatenate([bxr, bxz, bxn]).reshape(1, 3 * D)
    bhcat = jnp.concatenate([bhr, bhz, bhn]).reshape(1, 3 * D)
    bfc2 = bfc.reshape(1, D)
    Wfcb = Wfc.astype(jnp.bfloat16)

    degs = _deg(col3)
    dinv2d = _dinv(degs.reshape(NC, NPAD // D, D))
    dinvb = jnp.broadcast_to(dinv2d.reshape(NPAD)[:N, None], (N, D))

    rb = 2000
    xT = x.transpose(1, 0, 2, 3)
    xs_all = _scale(xT, dinvb, rb)

    Pxs = [_prop(xs_all[t].reshape(B * N, D), rowb, col3).reshape(B, N, D)
           for t in range(T)]

    h, hs, ot = _gate0(Pxs[0], dinvb, Wxcat, bxcat, bhcat, Wfcb, bfc2, rb)
    outs = [ot]
    for t in range(1, T):
        Ph = _prop(hs.reshape(B * N, D), rowb, col3).reshape(B, N, D)
        h, hs, ot = _gate(Pxs[t], Ph, h, dinvb, Wxcat, bxcat, Whcat, bhcat,
                          Wfcb, bfc2, rb)
        outs.append(ot)
    return jnp.stack(outs, axis=1)


# speedup vs baseline: 1.0015x; 1.0015x over previous
import jax
import jax.numpy as jnp
from jax import lax
from jax.experimental import pallas as pl
from jax.experimental.pallas import tpu as pltpu
from jax.experimental.pallas import tpu_sc as plsc

B, T, N, E = 2, 4, 10000, 320000
D = 128
NS = 16
NC = 2
NPAD = 10240
ZCH = NPAD // NS
CK = 80
CHT = 250
EPAD = NS * CHT * CK
NBUF = 4
NTRASH = 10000

_mesh = plsc.VectorSubcoreMesh(core_axis_name="c", subcore_axis_name="s")


DNB = 5


def _deg_body(col_hbm, deg_hbm, c0, c1, c2, c3, c4, c5, onesv, zbuf,
              si0, si1, si2, si3, si4, si5,
              ss0, ss1, ss2, ss3, ss4, ss5, deg_sh):
    c = lax.axis_index("c")
    s = lax.axis_index("s")
    colv = (c0, c1, c2, c3, c4, c5)
    si = (si0, si1, si2, si3, si4, si5)
    ss = (ss0, ss1, ss2, ss3, ss4, ss5)
    for j in range(ZCH // 16):
        zbuf[pl.ds(j * 16, 16)] = jnp.zeros((16,), jnp.float32)
    for j in range(CK // 16):
        onesv[pl.ds(j * 16, 16)] = jnp.ones((16,), jnp.float32)
    onesv[pl.ds(CK - 16, 16)] = jnp.ones((16,), jnp.float32)
    pltpu.sync_copy(zbuf, deg_sh.at[pl.ds(s * ZCH, ZCH)])
    plsc.subcore_barrier()
    ept = EPAD // (NS * NC)
    base = (s * NC + c) * ept

    def block(outer, carry):
        idp = []
        for b in range(DNB):
            @pl.when(outer > 0)
            def _():
                pltpu.make_async_copy(onesv.at[pl.ds(0, CK)],
                                      deg_sh.at[colv[b]], ss[b]).wait()

            off = base + (outer * DNB + b) * CK
            idp.append(pltpu.async_copy(col_hbm.at[pl.ds(off, CK)],
                                        colv[b], si[b]))
        for b in range(DNB):
            idp[b].wait()
            pltpu.async_copy(onesv.at[pl.ds(0, CK)], deg_sh.at[colv[b]],
                             ss[b], add=True)
        return carry

    lax.fori_loop(0, ept // (CK * DNB), block, 0)
    for b in range(DNB):
        pltpu.make_async_copy(onesv.at[pl.ds(0, CK)], deg_sh.at[colv[b]],
                              ss[b]).wait()
    plsc.subcore_barrier()
    pltpu.sync_copy(deg_sh.at[pl.ds(s * ZCH, ZCH)],
                    deg_hbm.at[pl.ds(c * NPAD + s * ZCH, ZCH)])


_deg = pl.kernel(
    _deg_body,
    out_type=jax.ShapeDtypeStruct((NC * NPAD,), jnp.float32),
    mesh=_mesh,
    scratch_types=(
        [pltpu.VMEM((CK,), jnp.int32) for _ in range(6)]
        + [pltpu.VMEM((CK,), jnp.float32), pltpu.VMEM((ZCH,), jnp.float32)]
        + [pltpu.SemaphoreType.DMA for _ in range(12)]
        + [pltpu.VMEM_SHARED((NPAD,), jnp.float32)]
    ),
)


def _make_prop(nt):
    def body(xs_hbm, rowb_hbm, col_hbm, out_hbm,
             r0, r1, r2, r3, c0, c1, c2, c3, g0, g1, g2, g3,
             si0, si1, si2, si3, sg0, sg1, sg2, sg3,
             ss0, ss1, ss2, ss3, out_sh):
        c = lax.axis_index("c")
        s = lax.axis_index("s")
        rowv = (r0, r1, r2, r3)
        colv = (c0, c1, c2, c3)
        gbuf = (g0, g1, g2, g3)
        si = (si0, si1, si2, si3)
        sg = (sg0, sg1, sg2, sg3)
        ss = (ss0, ss1, ss2, ss3)
        rpt = N // 10
        ept = CHT * CK
        base = s * ept

        for t in range(nt):
            @pl.when(s < 10)
            def _():
                pltpu.sync_copy(
                    xs_hbm.at[pl.ds(t * B * N + c * N + s * rpt, rpt)],
                    out_sh.at[pl.ds(s * rpt, rpt)])

            plsc.subcore_barrier()
            ibase = (t * NC + c) * EPAD + base

            def block(outer, carry):
                idp = []
                for b in range(NBUF):
                    @pl.when(outer > 0)
                    def _():
                        pltpu.make_async_copy(gbuf[b], out_sh.at[colv[b]],
                                              ss[b]).wait()

                    off = (outer * NBUF + b) * CK
                    i1 = pltpu.async_copy(
                        rowb_hbm.at[pl.ds(ibase + off, CK)], rowv[b], si[b])
                    i2 = pltpu.async_copy(
                        col_hbm.at[pl.ds(base + off, CK)], colv[b], si[b])
                    idp.append((i1, i2))
                gd = []
                for b in range(NBUF):
                    idp[b][0].wait()
                    idp[b][1].wait()
                    gd.append(pltpu.async_copy(xs_hbm.at[rowv[b]], gbuf[b],
                                               sg[b]))
                for b in range(NBUF):
                    gd[b].wait()
                    pltpu.async_copy(gbuf[b], out_sh.at[colv[b]], ss[b],
                                     add=True)
                return carry

            lax.fori_loop(0, CHT // NBUF, block, 0)
            for k in range(CHT - NBUF * (CHT // NBUF)):
                pltpu.make_async_copy(gbuf[k], out_sh.at[colv[k]],
                                      ss[k]).wait()
                off = (NBUF * (CHT // NBUF) + k) * CK
                pltpu.sync_copy(rowb_hbm.at[pl.ds(ibase + off, CK)], rowv[k])
                pltpu.sync_copy(col_hbm.at[pl.ds(base + off, CK)], colv[k])
                pltpu.sync_copy(xs_hbm.at[rowv[k]], gbuf[k])
                pltpu.async_copy(gbuf[k], out_sh.at[colv[k]], ss[k], add=True)
            for b in range(NBUF):
                pltpu.make_async_copy(gbuf[b], out_sh.at[colv[b]],
                                      ss[b]).wait()
            plsc.subcore_barrier()

            @pl.when(s < 10)
            def _():
                pltpu.sync_copy(
                    out_sh.at[pl.ds(s * rpt, rpt)],
                    out_hbm.at[pl.ds(t * B * N + c * N + s * rpt, rpt)])

    return pl.kernel(
        body,
        out_type=jax.ShapeDtypeStruct((nt * B * N, D), jnp.float32),
        mesh=_mesh,
        scratch_types=(
            [pltpu.VMEM((CK,), jnp.int32) for _ in range(NBUF)]
            + [pltpu.VMEM((CK,), jnp.int32) for _ in range(NBUF)]
            + [pltpu.VMEM((CK, D), jnp.float32) for _ in range(NBUF)]
            + [pltpu.SemaphoreType.DMA for _ in range(3 * NBUF)]
            + [pltpu.VMEM_SHARED((NTRASH, D), jnp.float32)]
        ),
    )


_prop = _make_prop(1)


def _dinv_body(deg_ref, out_ref):
    out_ref[...] = lax.rsqrt(deg_ref[0] + deg_ref[1] + 1.0)


def _dinv(degs):
    return pl.pallas_call(
        _dinv_body,
        out_shape=jax.ShapeDtypeStruct((NPAD // D, D), jnp.float32),
    )(degs)


def _scale_body(x_ref, d_ref, o_ref):
    o_ref[0, 0] = x_ref[0, 0] * d_ref[...]


def _scale(xT, dinvb, rb):
    return pl.pallas_call(
        _scale_body,
        grid=(T, B, N // rb),
        in_specs=[
            pl.BlockSpec((1, 1, rb, D), lambda t, b, i: (t, b, i, 0)),
            pl.BlockSpec((rb, D), lambda t, b, i: (i, 0)),
        ],
        out_specs=pl.BlockSpec((1, 1, rb, D), lambda t, b, i: (t, b, i, 0)),
        out_shape=jax.ShapeDtypeStruct((T, B, N, D), jnp.float32),
    )(xT, dinvb)


def _gate0_body(px, dv, wx, bx, bh, wf, bf, hn_o, hs_o, out_o):
    d = dv[...]
    u = jnp.dot(px[0] * d, wx[...], preferred_element_type=jnp.float32) + bx[...]
    v = bh[...]
    r = jax.nn.sigmoid(u[:, :D] + v[:, :D])
    z = jax.nn.sigmoid(u[:, D:2 * D] + v[:, D:2 * D])
    n = jnp.tanh(u[:, 2 * D:] + r * v[:, 2 * D:])
    hn = z * n
    hn_o[0] = hn
    hs_o[0] = hn * d
    out_o[0] = jnp.dot(hn, wf[...], preferred_element_type=jnp.float32) + bf[...]


def _gate0(Px, dinvb, Wx, bx, bh, Wf, bf, rb):
    node = pl.BlockSpec((1, rb, D), lambda b, i: (b, i, 0))
    return pl.pallas_call(
        _gate0_body,
        grid=(B, N // rb),
        in_specs=[
            node,
            pl.BlockSpec((rb, D), lambda b, i: (i, 0)),
            pl.BlockSpec((D, 3 * D), lambda b, i: (0, 0)),
            pl.BlockSpec((1, 3 * D), lambda b, i: (0, 0)),
            pl.BlockSpec((1, 3 * D), lambda b, i: (0, 0)),
            pl.BlockSpec((D, D), lambda b, i: (0, 0)),
            pl.BlockSpec((1, D), lambda b, i: (0, 0)),
        ],
        out_specs=[node, node, node],
        out_shape=[
            jax.ShapeDtypeStruct((B, N, D), jnp.float32),
            jax.ShapeDtypeStruct((B, N, D), jnp.float32),
            jax.ShapeDtypeStruct((B, N, D), jnp.float32),
        ],
    )(Px, dinvb, Wx, bx, bh, Wf, bf)


def _gate_body(px, ph, h, dv, wx, bx, wh, bh, wf, bf, hn_o, hs_o, out_o):
    d = dv[...]
    u = jnp.dot(px[0] * d, wx[...], preferred_element_type=jnp.float32) + bx[...]
    v = jnp.dot(ph[0] * d, wh[...], preferred_element_type=jnp.float32) + bh[...]
    r = jax.nn.sigmoid(u[:, :D] + v[:, :D])
    z = jax.nn.sigmoid(u[:, D:2 * D] + v[:, D:2 * D])
    n = jnp.tanh(u[:, 2 * D:] + r * v[:, 2 * D:])
    hn = (1.0 - z) * h[0] + z * n
    hn_o[0] = hn
    hs_o[0] = hn * d
    out_o[0] = jnp.dot(hn, wf[...], preferred_element_type=jnp.float32) + bf[...]


def _gate(Px, Ph, h, dinvb, Wx, bx, Wh, bh, Wf, bf, rb):
    node = pl.BlockSpec((1, rb, D), lambda b, i: (b, i, 0))
    return pl.pallas_call(
        _gate_body,
        grid=(B, N // rb),
        in_specs=[
            node, node, node,
            pl.BlockSpec((rb, D), lambda b, i: (i, 0)),
            pl.BlockSpec((D, 3 * D), lambda b, i: (0, 0)),
            pl.BlockSpec((1, 3 * D), lambda b, i: (0, 0)),
            pl.BlockSpec((D, 3 * D), lambda b, i: (0, 0)),
            pl.BlockSpec((1, 3 * D), lambda b, i: (0, 0)),
            pl.BlockSpec((D, D), lambda b, i: (0, 0)),
            pl.BlockSpec((1, D), lambda b, i: (0, 0)),
        ],
        out_specs=[node, node, node],
        out_shape=[
            jax.ShapeDtypeStruct((B, N, D), jnp.float32),
            jax.ShapeDtypeStruct((B, N, D), jnp.float32),
            jax.ShapeDtypeStruct((B, N, D), jnp.float32),
        ],
    )(Px, Ph, h, dinvb, Wx, bx, Wh, bh, Wf, bf)


def kernel(x, edge_index, Wxr, bxr, Wxz, bxz, Wxn, bxn,
           Whr, bhr, Whz, bhz, Whn, bhn, Wfc, bfc):
    row = edge_index[0]
    col = edge_index[1]
    rowb = jnp.concatenate([row, row + N])
    col3 = col
    Wxcat = jnp.concatenate([Wxr, Wxz, Wxn], axis=1)
    Whcat = jnp.concatenate([Whr, Whz, Whn], axis=1)
    bxcat = jnp.concatenate([bxr, bxz, bxn]).reshape(1, 3 * D)
    bhcat = jnp.concatenate([bhr, bhz, bhn]).reshape(1, 3 * D)
    bfc2 = bfc.reshape(1, D)

    degs = _deg(col3)
    dinv2d = _dinv(degs.reshape(NC, NPAD // D, D))
    dinvb = jnp.broadcast_to(dinv2d.reshape(NPAD)[:N, None], (N, D))

    rb = 2000
    xT = x.transpose(1, 0, 2, 3)
    xs_all = _scale(xT, dinvb, rb)

    Pxs = [_prop(xs_all[t].reshape(B * N, D), rowb, col3).reshape(B, N, D)
           for t in range(T)]

    h, hs, ot = _gate0(Pxs[0], dinvb, Wxcat, bxcat, bhcat, Wfc, bfc2, rb)
    outs = [ot]
    for t in range(1, T):
        Ph = _prop(hs.reshape(B * N, D), rowb, col3).reshape(B, N, D)
        h, hs, ot = _gate(Pxs[t], Ph, h, dinvb, Wxcat, bxcat, Whcat, bhcat,
                          Wfc, bfc2, rb)
        outs.append(ot)
    return jnp.stack(outs, axis=1)
